# Initial kernel scaffold; baseline (speedup 1.0000x reference)
import jax
import jax.numpy as jnp
from jax.experimental import pallas as pl


def kernel(preds_S, preds_T, depth_gt_resized):
    raise NotImplementedError("write your pallas kernel here")


# speedup vs baseline: 1.0911x; 1.0911x over previous
import functools

import jax
import jax.numpy as jnp
from jax import lax
from jax.experimental import pallas as pl
from jax.experimental.pallas import tpu as pltpu
from jax.experimental.pallas import tpu_sc as plsc

NBINS = 64
NSEG = NBINS + 1
EPS = 1e-12
NC, NS = 2, 16
NW = NC * NS
LANES = 16
HIST = NSEG * LANES


def _sc_segment_sums(sres, tres, dres):
    N, C, HW = sres.shape
    CPL = C // NW
    QB = 3200
    assert HW % QB == 0
    QCH = HW // QB

    mesh = plsc.VectorSubcoreMesh(core_axis_name="c", subcore_axis_name="s")

    @functools.partial(
        pl.kernel,
        out_type=(
            jax.ShapeDtypeStruct((NW, CPL * HIST), jnp.float32),
            jax.ShapeDtypeStruct((NW, CPL * HIST), jnp.float32),
            jax.ShapeDtypeStruct((NW, HIST), jnp.float32),
        ),
        mesh=mesh,
        compiler_params=pltpu.CompilerParams(
            use_tc_tiling_on_sc=False, needs_layout_passes=False),
        scratch_types=[
            pltpu.VMEM((CPL, QB), jnp.float32),
            pltpu.VMEM((CPL, QB), jnp.float32),
            pltpu.VMEM((QB,), jnp.float32),
            pltpu.VMEM((CPL * HIST,), jnp.float32),
            pltpu.VMEM((CPL * HIST,), jnp.float32),
            pltpu.VMEM((HIST,), jnp.float32),
        ],
    )
    def seg_sum(sref, tref, dref, out_s, out_t, out_c,
                buf_s, buf_t, dv, hist_s, hist_t, hist_c):
        cid_ax = lax.axis_index("c")
        sid_ax = lax.axis_index("s")
        w = sid_ax * NC + cid_ax
        cbase = w * CPL

        zero = jnp.zeros((LANES,), jnp.float32)
        one = jnp.ones((LANES,), jnp.float32)
        lane = lax.iota(jnp.int32, LANES)

        def zh(i, carry):
            hist_s[pl.ds(i * LANES, LANES)] = zero
            hist_t[pl.ds(i * LANES, LANES)] = zero
            return carry

        lax.fori_loop(0, CPL * NSEG, zh, 0)

        def zc(i, carry):
            hist_c[pl.ds(i * LANES, LANES)] = zero
            return carry

        lax.fori_loop(0, NSEG, zc, 0)

        for n in range(N):
            def chunk(qc, carry):
                q0 = qc * QB
                pltpu.sync_copy(dref.at[pl.ds(n * HW + q0, QB)], dv)
                pltpu.sync_copy(sref.at[n, pl.ds(cbase, CPL), pl.ds(q0, QB)],
                                buf_s)
                pltpu.sync_copy(tref.at[n, pl.ds(cbase, CPL), pl.ds(q0, QB)],
                                buf_t)

                def group(g, carry2):
                    d = dv[pl.ds(g * LANES, LANES)]
                    f = d * jnp.float32(NBINS)
                    bad = ((f < 0.0) | (f > jnp.float32(NBINS))
                           | ((d - d) != 0.0))
                    seg = jnp.where(bad, jnp.float32(NBINS), f)
                    iv = seg.astype(jnp.int32) * LANES + lane
                    plsc.addupdate_scatter(hist_c, [iv], one)
                    for c in range(CPL):
                        ivc = iv + c * HIST
                        plsc.addupdate_scatter(
                            hist_s, [ivc], buf_s[c, pl.ds(g * LANES, LANES)])
                        plsc.addupdate_scatter(
                            hist_t, [ivc], buf_t[c, pl.ds(g * LANES, LANES)])
                    return carry2

                lax.fori_loop(0, QB // LANES, group, 0)
                return carry

            lax.fori_loop(0, QCH, chunk, 0)

        pltpu.sync_copy(hist_s, out_s.at[w])
        pltpu.sync_copy(hist_t, out_t.at[w])
        pltpu.sync_copy(hist_c, out_c.at[w])

    return seg_sum(sres, tres, dres)


def _finalize_body(acc_s_ref, acc_t_ref, acc_c_ref, out_ref):
    row = lax.broadcasted_iota(jnp.int32, (HIST, NSEG), 0) // LANES
    col = lax.broadcasted_iota(jnp.int32, (HIST, NSEG), 1)
    lane_sum = (row == col).astype(jnp.float32)
    dims_rc = (((1,), (0,)), ((), ()))
    hi = lax.Precision.HIGHEST

    counts = lax.dot_general(
        acc_c_ref[...][0:1, :], lane_sum, dims_rc, precision=hi)
    cnt = jnp.maximum(counts[:, :NBINS], 1.0)

    def feats(acc):
        sums = lax.dot_general(acc[...], lane_sum, dims_rc, precision=hi)
        means = sums[:, :NBINS] / cnt
        nrm = jnp.sqrt(jnp.sum(means * means, axis=0, keepdims=True))
        return means / jnp.maximum(nrm, EPS)

    f_s = feats(acc_s_ref)
    f_t = feats(acc_t_ref)
    dims_cc = (((0,), (0,)), ((), ()))
    sim_s = lax.dot_general(f_s, f_s, dims_cc, precision=hi)
    sim_t = lax.dot_general(f_t, f_t, dims_cc, precision=hi)
    diff = sim_s - sim_t
    out_ref[0, 0] = jnp.mean(diff * diff)


def kernel(preds_S, preds_T, depth_gt_resized):
    N, C, H, W = preds_S.shape
    sres = preds_S.reshape(N, C, H * W)
    tres = preds_T.reshape(N, C, H * W)
    dres = depth_gt_resized.reshape(N * H * W)
    acc_s, acc_t, acc_c = _sc_segment_sums(sres, tres, dres)
    acc_s = acc_s.reshape(C, HIST)
    acc_t = acc_t.reshape(C, HIST)
    loss = pl.pallas_call(
        _finalize_body,
        out_shape=jax.ShapeDtypeStruct((1, 1), jnp.float32),
        out_specs=pl.BlockSpec(memory_space=pltpu.SMEM),
    )(acc_s, acc_t, acc_c)
    return loss[0, 0]


# speedup vs baseline: 1.2617x; 1.1564x over previous
import functools

import jax
import jax.numpy as jnp
from jax import lax
from jax.experimental import pallas as pl
from jax.experimental.pallas import tpu as pltpu
from jax.experimental.pallas import tpu_sc as plsc

NBINS = 64
NSEG = NBINS + 1
EPS = 1e-12
NC, NS = 2, 16
NW = NC * NS
LANES = 16
HIST = NSEG * LANES


def _sc_segment_sums(sres, tres, dres):
    N, C, HW = sres.shape
    CPL = C // NW
    QB = 3200
    assert HW % QB == 0
    QCH = HW // QB

    mesh = plsc.VectorSubcoreMesh(core_axis_name="c", subcore_axis_name="s")

    @functools.partial(
        pl.kernel,
        out_type=(
            jax.ShapeDtypeStruct((NW, CPL * HIST), jnp.float32),
            jax.ShapeDtypeStruct((NW, CPL * HIST), jnp.float32),
            jax.ShapeDtypeStruct((NW, HIST), jnp.float32),
        ),
        mesh=mesh,
        compiler_params=pltpu.CompilerParams(needs_layout_passes=False),
        scratch_types=[
            pltpu.VMEM((CPL, QB), jnp.float32),
            pltpu.VMEM((CPL, QB), jnp.float32),
            pltpu.VMEM((QB,), jnp.float32),
            pltpu.VMEM((CPL * HIST,), jnp.float32),
            pltpu.VMEM((CPL * HIST,), jnp.float32),
            pltpu.VMEM((HIST,), jnp.float32),
        ],
    )
    def seg_sum(sref, tref, dref, out_s, out_t, out_c,
                buf_s, buf_t, dv, hist_s, hist_t, hist_c):
        cid_ax = lax.axis_index("c")
        sid_ax = lax.axis_index("s")
        w = sid_ax * NC + cid_ax
        cbase = w * CPL

        zero = jnp.zeros((LANES,), jnp.float32)
        one = jnp.ones((LANES,), jnp.float32)
        lane = lax.iota(jnp.int32, LANES)

        def zh(i, carry):
            hist_s[pl.ds(i * LANES, LANES)] = zero
            hist_t[pl.ds(i * LANES, LANES)] = zero
            return carry

        lax.fori_loop(0, CPL * NSEG, zh, 0)

        def zc(i, carry):
            hist_c[pl.ds(i * LANES, LANES)] = zero
            return carry

        lax.fori_loop(0, NSEG, zc, 0)

        for n in range(N):
            def chunk(qc, carry):
                q0 = qc * QB
                pltpu.sync_copy(dref.at[pl.ds(n * HW + q0, QB)], dv)
                pltpu.sync_copy(sref.at[n, pl.ds(cbase, CPL), pl.ds(q0, QB)],
                                buf_s)
                pltpu.sync_copy(tref.at[n, pl.ds(cbase, CPL), pl.ds(q0, QB)],
                                buf_t)

                def group(g, carry2):
                    d = dv[pl.ds(g * LANES, LANES)]
                    f = d * jnp.float32(NBINS)
                    bad = ((f < 0.0) | (f > jnp.float32(NBINS))
                           | ((d - d) != 0.0))
                    seg = jnp.where(bad, jnp.float32(NBINS), f)
                    iv = seg.astype(jnp.int32) * LANES + lane
                    plsc.addupdate_scatter(hist_c, [iv], one)
                    for c in range(CPL):
                        ivc = iv + c * HIST
                        plsc.addupdate_scatter(
                            hist_s, [ivc], buf_s[c, pl.ds(g * LANES, LANES)])
                        plsc.addupdate_scatter(
                            hist_t, [ivc], buf_t[c, pl.ds(g * LANES, LANES)])
                    return carry2

                lax.fori_loop(0, QB // LANES, group, 0)
                return carry

            lax.fori_loop(0, QCH, chunk, 0)

        pltpu.sync_copy(hist_s, out_s.at[w])
        pltpu.sync_copy(hist_t, out_t.at[w])
        pltpu.sync_copy(hist_c, out_c.at[w])

    return seg_sum(sres, tres, dres)


def _finalize_body(acc_s_ref, acc_t_ref, acc_c_ref, out_ref):
    row = lax.broadcasted_iota(jnp.int32, (HIST, NSEG), 0) // LANES
    col = lax.broadcasted_iota(jnp.int32, (HIST, NSEG), 1)
    lane_sum = (row == col).astype(jnp.float32)
    dims_rc = (((1,), (0,)), ((), ()))
    hi = lax.Precision.HIGHEST

    counts = lax.dot_general(
        acc_c_ref[...][0:1, :], lane_sum, dims_rc, precision=hi)
    cnt = jnp.maximum(counts[:, :NBINS], 1.0)

    def feats(acc):
        sums = lax.dot_general(acc[...], lane_sum, dims_rc, precision=hi)
        means = sums[:, :NBINS] / cnt
        nrm = jnp.sqrt(jnp.sum(means * means, axis=0, keepdims=True))
        return means / jnp.maximum(nrm, EPS)

    f_s = feats(acc_s_ref)
    f_t = feats(acc_t_ref)
    dims_cc = (((0,), (0,)), ((), ()))
    sim_s = lax.dot_general(f_s, f_s, dims_cc, precision=hi)
    sim_t = lax.dot_general(f_t, f_t, dims_cc, precision=hi)
    diff = sim_s - sim_t
    out_ref[0, 0] = jnp.mean(diff * diff)


def kernel(preds_S, preds_T, depth_gt_resized):
    N, C, H, W = preds_S.shape
    sres = preds_S.reshape(N, C, H * W)
    tres = preds_T.reshape(N, C, H * W)
    dres = depth_gt_resized.reshape(N * H * W)
    acc_s, acc_t, acc_c = _sc_segment_sums(sres, tres, dres)
    acc_s = acc_s.reshape(C, HIST)
    acc_t = acc_t.reshape(C, HIST)
    loss = pl.pallas_call(
        _finalize_body,
        out_shape=jax.ShapeDtypeStruct((1, 1), jnp.float32),
        out_specs=pl.BlockSpec(memory_space=pltpu.SMEM),
    )(acc_s, acc_t, acc_c)
    return loss[0, 0]


# speedup vs baseline: 2.7360x; 2.1685x over previous
import functools

import jax
import jax.numpy as jnp
from jax import lax
from jax.experimental import pallas as pl
from jax.experimental.pallas import tpu as pltpu
from jax.experimental.pallas import tpu_sc as plsc

NBINS = 64
NSEG = NBINS + 1
EPS = 1e-12
NC, NS = 2, 16
NW = NC * NS
LANES = 16
HIST = NSEG * LANES


def _sc_segment_sums(sres, tres, dres):
    N, C, HW = sres.shape
    CPL = C // NW
    QB = 3200
    assert HW % QB == 0
    QCH = HW // QB

    mesh = plsc.VectorSubcoreMesh(core_axis_name="c", subcore_axis_name="s")

    @functools.partial(
        pl.kernel,
        out_type=(
            jax.ShapeDtypeStruct((NW, CPL * HIST), jnp.float32),
            jax.ShapeDtypeStruct((NW, CPL * HIST), jnp.float32),
            jax.ShapeDtypeStruct((NW, HIST), jnp.float32),
        ),
        mesh=mesh,
        compiler_params=pltpu.CompilerParams(needs_layout_passes=False),
        scratch_types=[
            pltpu.VMEM((CPL, QB), jnp.float32),
            pltpu.VMEM((CPL, QB), jnp.float32),
            pltpu.VMEM((QB,), jnp.float32),
            pltpu.VMEM((QB,), jnp.int32),
            pltpu.VMEM((CPL * HIST,), jnp.float32),
            pltpu.VMEM((CPL * HIST,), jnp.float32),
            pltpu.VMEM((HIST,), jnp.float32),
        ],
    )
    def seg_sum(sref, tref, dref, out_s, out_t, out_c,
                buf_s, buf_t, dv, ivbuf, hist_s, hist_t, hist_c):
        cid_ax = lax.axis_index("c")
        sid_ax = lax.axis_index("s")
        w = sid_ax * NC + cid_ax
        cbase = w * CPL

        zero = jnp.zeros((LANES,), jnp.float32)
        one = jnp.ones((LANES,), jnp.float32)
        lane = lax.iota(jnp.int32, LANES)

        def zh(i, carry):
            hist_s[pl.ds(i * LANES, LANES)] = zero
            hist_t[pl.ds(i * LANES, LANES)] = zero
            return carry

        lax.fori_loop(0, CPL * NSEG, zh, 0)

        def zc(i, carry):
            hist_c[pl.ds(i * LANES, LANES)] = zero
            return carry

        lax.fori_loop(0, NSEG, zc, 0)

        for n in range(N):
            def chunk(qc, carry):
                q0 = qc * QB
                pltpu.sync_copy(dref.at[pl.ds(n * HW + q0, QB)], dv)
                pltpu.sync_copy(sref.at[n, pl.ds(cbase, CPL), pl.ds(q0, QB)],
                                buf_s)
                pltpu.sync_copy(tref.at[n, pl.ds(cbase, CPL), pl.ds(q0, QB)],
                                buf_t)

                @functools.partial(
                    plsc.parallel_loop, 0, QB // LANES, unroll=4)
                def _(g):
                    d = dv[pl.ds(g * LANES, LANES)]
                    f = d * jnp.float32(NBINS)
                    bad = ((f < 0.0) | (f > jnp.float32(NBINS))
                           | ((d - d) != 0.0))
                    seg = jnp.where(bad, jnp.float32(NBINS), f)
                    ivbuf[pl.ds(g * LANES, LANES)] = (
                        seg.astype(jnp.int32) * LANES + lane)

                @functools.partial(
                    plsc.parallel_loop, 0, QB // LANES, unroll=2)
                def _(g):
                    iv = ivbuf[pl.ds(g * LANES, LANES)]
                    plsc.addupdate_scatter(hist_c, [iv], one)
                    for c in range(CPL):
                        ivc = iv + c * HIST
                        plsc.addupdate_scatter(
                            hist_s, [ivc], buf_s[c, pl.ds(g * LANES, LANES)])
                        plsc.addupdate_scatter(
                            hist_t, [ivc], buf_t[c, pl.ds(g * LANES, LANES)])

                return carry

            lax.fori_loop(0, QCH, chunk, 0)

        pltpu.sync_copy(hist_s, out_s.at[w])
        pltpu.sync_copy(hist_t, out_t.at[w])
        pltpu.sync_copy(hist_c, out_c.at[w])

    return seg_sum(sres, tres, dres)


def _finalize_body(acc_s_ref, acc_t_ref, acc_c_ref, out_ref):
    row = lax.broadcasted_iota(jnp.int32, (HIST, NSEG), 0) // LANES
    col = lax.broadcasted_iota(jnp.int32, (HIST, NSEG), 1)
    lane_sum = (row == col).astype(jnp.float32)
    dims_rc = (((1,), (0,)), ((), ()))
    hi = lax.Precision.HIGHEST

    counts = lax.dot_general(
        acc_c_ref[...][0:1, :], lane_sum, dims_rc, precision=hi)
    cnt = jnp.maximum(counts[:, :NBINS], 1.0)

    def feats(acc):
        sums = lax.dot_general(acc[...], lane_sum, dims_rc, precision=hi)
        means = sums[:, :NBINS] / cnt
        nrm = jnp.sqrt(jnp.sum(means * means, axis=0, keepdims=True))
        return means / jnp.maximum(nrm, EPS)

    f_s = feats(acc_s_ref)
    f_t = feats(acc_t_ref)
    dims_cc = (((0,), (0,)), ((), ()))
    sim_s = lax.dot_general(f_s, f_s, dims_cc, precision=hi)
    sim_t = lax.dot_general(f_t, f_t, dims_cc, precision=hi)
    diff = sim_s - sim_t
    out_ref[0, 0] = jnp.mean(diff * diff)


def kernel(preds_S, preds_T, depth_gt_resized):
    N, C, H, W = preds_S.shape
    sres = preds_S.reshape(N, C, H * W)
    tres = preds_T.reshape(N, C, H * W)
    dres = depth_gt_resized.reshape(N * H * W)
    acc_s, acc_t, acc_c = _sc_segment_sums(sres, tres, dres)
    acc_s = acc_s.reshape(C, HIST)
    acc_t = acc_t.reshape(C, HIST)
    loss = pl.pallas_call(
        _finalize_body,
        out_shape=jax.ShapeDtypeStruct((1, 1), jnp.float32),
        out_specs=pl.BlockSpec(memory_space=pltpu.SMEM),
    )(acc_s, acc_t, acc_c)
    return loss[0, 0]
